# baseline (speedup 1.0000x reference)
import functools

import jax
import jax.numpy as jnp
from jax import lax
from jax.experimental import pallas as pl
from jax.experimental.pallas import tpu as pltpu
from jax.experimental.pallas import tpu_sc as plsc

HIDDEN_DIM = 768
NUM_EXPERTS = 8
TOKENS = 32768
BLOCK_T = 2048
NUM_CHUNKS = 1
CHUNK_T = TOKENS // NUM_CHUNKS

_NUM_WORKERS = 32
_TOK_PER_W = CHUNK_T // _NUM_WORKERS
_GROUPS = _TOK_PER_W // 16


def _logits_body(x_ref, w_ref, b_ref, p_ref, idx_ref):
    x = x_ref[...]
    w = w_ref[...]
    lt = jax.lax.dot_general(w, x, (((1,), (1,)), ((), ())),
                             preferred_element_type=jnp.float32)
    lt = lt + b_ref[...]

    le = [lt[e:e + 1, :] for e in range(NUM_EXPERTS)]
    v1 = le[0]
    i1 = jnp.zeros(v1.shape, jnp.int32)
    for e in range(1, NUM_EXPERTS):
        gt = le[e] > v1
        v1 = jnp.where(gt, le[e], v1)
        i1 = jnp.where(gt, e, i1)
    nz = i1 != 0
    v2 = jnp.where(nz, le[0], le[1])
    i2 = jnp.where(nz, 0, 1)
    for e in range(1, NUM_EXPERTS):
        gt = (le[e] > v2) & (i1 != e)
        v2 = jnp.where(gt, le[e], v2)
        i2 = jnp.where(gt, e, i2)

    d = jnp.exp(v2 - v1)
    p1 = 1.0 / (1.0 + d)
    p_ref[...] = jnp.concatenate([p1, 1.0 - p1], axis=0)
    idx_ref[...] = jnp.concatenate([i1, i2], axis=0)


def _tc_logits(x, weight, b2):
    t = x.shape[0]
    grid = (t // BLOCK_T,)
    return pl.pallas_call(
        _logits_body,
        grid=grid,
        in_specs=[
            pl.BlockSpec((BLOCK_T, HIDDEN_DIM), lambda i: (i, 0)),
            pl.BlockSpec((NUM_EXPERTS, HIDDEN_DIM), lambda i: (0, 0)),
            pl.BlockSpec((NUM_EXPERTS, 1), lambda i: (0, 0)),
        ],
        out_specs=[
            pl.BlockSpec((2, BLOCK_T), lambda i: (0, i)),
            pl.BlockSpec((2, BLOCK_T), lambda i: (0, i)),
        ],
        out_shape=[
            jax.ShapeDtypeStruct((2, t), jnp.float32),
            jax.ShapeDtypeStruct((2, t), jnp.int32),
        ],
        compiler_params=pltpu.CompilerParams(
            dimension_semantics=("parallel",)),
    )(x, weight, b2)


def _route_body(p_hbm, idx_hbm, scores_hbm, pbuf, ibuf, scores_v):
    wid = lax.axis_index("s") * 2 + lax.axis_index("c")
    base = wid * _TOK_PER_W
    pltpu.sync_copy(p_hbm.at[:, pl.ds(base, _TOK_PER_W)], pbuf)
    pltpu.sync_copy(idx_hbm.at[:, pl.ds(base, _TOK_PER_W)], ibuf)

    zeros64 = jnp.zeros((16,), jnp.float32)

    def _zero(i, c):
        scores_v[pl.ds(i * 64, 16)] = zeros64
        scores_v[pl.ds(i * 64 + 16, 16)] = zeros64
        scores_v[pl.ds(i * 64 + 32, 16)] = zeros64
        scores_v[pl.ds(i * 64 + 48, 16)] = zeros64
        return c

    lax.fori_loop(0, _TOK_PER_W * NUM_EXPERTS // 64, _zero, 0)

    lane = lax.iota(jnp.int32, 16)

    def _group(g, c):
        p1 = pbuf[0, pl.ds(g * 16, 16)]
        p2 = pbuf[1, pl.ds(g * 16, 16)]
        i1 = ibuf[0, pl.ds(g * 16, 16)]
        i2 = ibuf[1, pl.ds(g * 16, 16)]
        tok = g * 16 + lane
        plsc.store_scatter(scores_v, [i1 * _TOK_PER_W + tok], p1)
        plsc.store_scatter(scores_v, [i2 * _TOK_PER_W + tok], p2)
        return c

    lax.fori_loop(0, _GROUPS, _group, 0)

    for e in range(NUM_EXPERTS):
        pltpu.sync_copy(
            scores_v.at[pl.ds(e * _TOK_PER_W, _TOK_PER_W)],
            scores_hbm.at[e, pl.ds(base, _TOK_PER_W)])


def _sc_route(p_t, idx_t):
    t = p_t.shape[1]
    mesh = plsc.VectorSubcoreMesh(core_axis_name="c", subcore_axis_name="s")
    run = pl.kernel(
        _route_body,
        out_type=jax.ShapeDtypeStruct((NUM_EXPERTS, t), jnp.float32),
        mesh=mesh,
        scratch_types=[
            pltpu.VMEM((2, _TOK_PER_W), jnp.float32),
            pltpu.VMEM((2, _TOK_PER_W), jnp.int32),
            pltpu.VMEM((_TOK_PER_W * NUM_EXPERTS,), jnp.float32),
        ],
        compiler_params=pltpu.CompilerParams(needs_layout_passes=False),
    )
    return run(p_t, idx_t)


@jax.jit
def kernel(hidden_states, weight, bias):
    x = hidden_states.reshape(-1, HIDDEN_DIM)
    b2 = bias.reshape(NUM_EXPERTS, 1)
    p_t, idx_t = _tc_logits(x, weight, b2)
    s_t = _sc_route(p_t, idx_t)
    return s_t.T, idx_t.T


# speedup vs baseline: 1.0058x; 1.0058x over previous
import functools

import jax
import jax.numpy as jnp
from jax import lax
from jax.experimental import pallas as pl
from jax.experimental.pallas import tpu as pltpu
from jax.experimental.pallas import tpu_sc as plsc

HIDDEN_DIM = 768
NUM_EXPERTS = 8
TOKENS = 32768
BLOCK_T = 2048
NUM_CHUNKS = 1
CHUNK_T = TOKENS // NUM_CHUNKS

_NUM_WORKERS = 32
_TOK_PER_W = CHUNK_T // _NUM_WORKERS
_GROUPS = _TOK_PER_W // 16


def _logits_body(x_ref, w_ref, b_ref, p_ref, idx_ref):
    x = x_ref[...]
    w = w_ref[...]
    lt = jax.lax.dot_general(w, x, (((1,), (1,)), ((), ())),
                             preferred_element_type=jnp.float32)
    lt = lt + b_ref[...]

    le = [lt[e:e + 1, :] for e in range(NUM_EXPERTS)]
    v1 = le[0]
    i1 = jnp.zeros(v1.shape, jnp.int32)
    for e in range(1, NUM_EXPERTS):
        gt = le[e] > v1
        v1 = jnp.where(gt, le[e], v1)
        i1 = jnp.where(gt, e, i1)
    nz = i1 != 0
    v2 = jnp.where(nz, le[0], le[1])
    i2 = jnp.where(nz, 0, 1)
    for e in range(1, NUM_EXPERTS):
        gt = (le[e] > v2) & (i1 != e)
        v2 = jnp.where(gt, le[e], v2)
        i2 = jnp.where(gt, e, i2)

    d = jnp.exp(v2 - v1)
    p1 = 1.0 / (1.0 + d)
    p_ref[...] = jnp.concatenate([p1, 1.0 - p1], axis=0)
    idx_ref[...] = jnp.concatenate([i1, i2], axis=0)


def _tc_logits(x, weight, b2):
    t = x.shape[0]
    grid = (t // BLOCK_T,)
    return pl.pallas_call(
        _logits_body,
        grid=grid,
        in_specs=[
            pl.BlockSpec((BLOCK_T, HIDDEN_DIM), lambda i: (i, 0)),
            pl.BlockSpec((NUM_EXPERTS, HIDDEN_DIM), lambda i: (0, 0)),
            pl.BlockSpec((NUM_EXPERTS, 1), lambda i: (0, 0)),
        ],
        out_specs=[
            pl.BlockSpec((2, BLOCK_T), lambda i: (0, i)),
            pl.BlockSpec((2, BLOCK_T), lambda i: (0, i)),
        ],
        out_shape=[
            jax.ShapeDtypeStruct((2, t), jnp.float32),
            jax.ShapeDtypeStruct((2, t), jnp.int32),
        ],
        compiler_params=pltpu.CompilerParams(
            dimension_semantics=("parallel",)),
    )(x, weight, b2)


def _route_body(p_hbm, idx_hbm, scores_hbm, pbuf, ibuf, scores_v):
    wid = lax.axis_index("s") * 2 + lax.axis_index("c")
    base = wid * _TOK_PER_W
    pltpu.sync_copy(p_hbm.at[:, pl.ds(base, _TOK_PER_W)], pbuf)
    pltpu.sync_copy(idx_hbm.at[:, pl.ds(base, _TOK_PER_W)], ibuf)

    zeros64 = jnp.zeros((16,), jnp.float32)

    def _zero(i, c):
        for e in range(NUM_EXPERTS):
            scores_v[e, pl.ds(i * 16, 16)] = zeros64
        return c

    lax.fori_loop(0, _TOK_PER_W // 16, _zero, 0)

    lane = lax.iota(jnp.int32, 16)

    def _group(g, c):
        p1 = pbuf[0, pl.ds(g * 16, 16)]
        p2 = pbuf[1, pl.ds(g * 16, 16)]
        i1 = ibuf[0, pl.ds(g * 16, 16)]
        i2 = ibuf[1, pl.ds(g * 16, 16)]
        tok = g * 16 + lane
        plsc.store_scatter(scores_v, [i1, tok], p1)
        plsc.store_scatter(scores_v, [i2, tok], p2)
        return c

    lax.fori_loop(0, _GROUPS, _group, 0)

    pltpu.sync_copy(scores_v, scores_hbm.at[:, pl.ds(base, _TOK_PER_W)])


def _sc_route(p_t, idx_t):
    t = p_t.shape[1]
    mesh = plsc.VectorSubcoreMesh(core_axis_name="c", subcore_axis_name="s")
    run = pl.kernel(
        _route_body,
        out_type=jax.ShapeDtypeStruct((NUM_EXPERTS, t), jnp.float32),
        mesh=mesh,
        scratch_types=[
            pltpu.VMEM((2, _TOK_PER_W), jnp.float32),
            pltpu.VMEM((2, _TOK_PER_W), jnp.int32),
            pltpu.VMEM((NUM_EXPERTS, _TOK_PER_W), jnp.float32),
        ],
        compiler_params=pltpu.CompilerParams(needs_layout_passes=False),
    )
    return run(p_t, idx_t)


@jax.jit
def kernel(hidden_states, weight, bias):
    x = hidden_states.reshape(-1, HIDDEN_DIM)
    b2 = bias.reshape(NUM_EXPERTS, 1)
    p_t, idx_t = _tc_logits(x, weight, b2)
    s_t = _sc_route(p_t, idx_t)
    return s_t.T, idx_t.T


# speedup vs baseline: 1.0254x; 1.0195x over previous
import functools

import jax
import jax.numpy as jnp
from jax import lax
from jax.experimental import pallas as pl
from jax.experimental.pallas import tpu as pltpu
from jax.experimental.pallas import tpu_sc as plsc

HIDDEN_DIM = 768
NUM_EXPERTS = 8
TOKENS = 32768
BLOCK_T = 2048
NUM_CHUNKS = 1
CHUNK_T = TOKENS // NUM_CHUNKS

_NUM_WORKERS = 32
_TOK_PER_W = CHUNK_T // _NUM_WORKERS
_GROUPS = _TOK_PER_W // 16


def _logits_body(x_ref, w_ref, b_ref, p_ref, idx_ref):
    x = x_ref[...]
    w = w_ref[...]
    lt = jax.lax.dot_general(w, x, (((1,), (1,)), ((), ())),
                             preferred_element_type=jnp.float32)
    lt = lt + b_ref[...]

    le = [lt[e:e + 1, :] for e in range(NUM_EXPERTS)]
    v1 = le[0]
    i1 = jnp.zeros(v1.shape, jnp.int32)
    for e in range(1, NUM_EXPERTS):
        gt = le[e] > v1
        v1 = jnp.where(gt, le[e], v1)
        i1 = jnp.where(gt, e, i1)
    nz = i1 != 0
    v2 = jnp.where(nz, le[0], le[1])
    i2 = jnp.where(nz, 0, 1)
    for e in range(1, NUM_EXPERTS):
        gt = (le[e] > v2) & (i1 != e)
        v2 = jnp.where(gt, le[e], v2)
        i2 = jnp.where(gt, e, i2)

    d = jnp.exp(v2 - v1)
    p1 = 1.0 / (1.0 + d)
    p_ref[...] = jnp.concatenate([p1, 1.0 - p1], axis=0)
    idx_ref[...] = jnp.concatenate([i1, i2], axis=0)


def _tc_logits(x, weight, b2):
    t = x.shape[0]
    grid = (t // BLOCK_T,)
    return pl.pallas_call(
        _logits_body,
        grid=grid,
        in_specs=[
            pl.BlockSpec((BLOCK_T, HIDDEN_DIM), lambda i: (i, 0)),
            pl.BlockSpec((NUM_EXPERTS, HIDDEN_DIM), lambda i: (0, 0)),
            pl.BlockSpec((NUM_EXPERTS, 1), lambda i: (0, 0)),
        ],
        out_specs=[
            pl.BlockSpec((2, BLOCK_T), lambda i: (0, i)),
            pl.BlockSpec((2, BLOCK_T), lambda i: (0, i)),
        ],
        out_shape=[
            jax.ShapeDtypeStruct((2, t), jnp.float32),
            jax.ShapeDtypeStruct((2, t), jnp.int32),
        ],
        compiler_params=pltpu.CompilerParams(
            dimension_semantics=("parallel",)),
    )(x, weight, b2)


def _route_body(p_hbm, idx_hbm, scores_hbm, pbuf, ibuf, scores_v, sem):
    wid = lax.axis_index("s") * 2 + lax.axis_index("c")
    base = wid * _TOK_PER_W
    cp = pltpu.async_copy(p_hbm.at[:, pl.ds(base, _TOK_PER_W)], pbuf, sem)
    ci = pltpu.async_copy(idx_hbm.at[:, pl.ds(base, _TOK_PER_W)], ibuf, sem)

    zeros64 = jnp.zeros((16,), jnp.float32)

    def _zero(i, c):
        for e in range(NUM_EXPERTS):
            scores_v[e, pl.ds(i * 16, 16)] = zeros64
        return c

    lax.fori_loop(0, _TOK_PER_W // 16, _zero, 0)
    cp.wait()
    ci.wait()

    lane = lax.iota(jnp.int32, 16)

    def _group(g, c):
        p1 = pbuf[0, pl.ds(g * 16, 16)]
        p2 = pbuf[1, pl.ds(g * 16, 16)]
        i1 = ibuf[0, pl.ds(g * 16, 16)]
        i2 = ibuf[1, pl.ds(g * 16, 16)]
        tok = g * 16 + lane
        plsc.store_scatter(scores_v, [i1, tok], p1)
        plsc.store_scatter(scores_v, [i2, tok], p2)
        return c

    lax.fori_loop(0, _GROUPS, _group, 0)

    pltpu.sync_copy(scores_v, scores_hbm.at[:, pl.ds(base, _TOK_PER_W)])


def _sc_route(p_t, idx_t):
    t = p_t.shape[1]
    mesh = plsc.VectorSubcoreMesh(core_axis_name="c", subcore_axis_name="s")
    run = pl.kernel(
        _route_body,
        out_type=jax.ShapeDtypeStruct((NUM_EXPERTS, t), jnp.float32),
        mesh=mesh,
        scratch_types=[
            pltpu.VMEM((2, _TOK_PER_W), jnp.float32),
            pltpu.VMEM((2, _TOK_PER_W), jnp.int32),
            pltpu.VMEM((NUM_EXPERTS, _TOK_PER_W), jnp.float32),
            pltpu.SemaphoreType.DMA,
        ],
        compiler_params=pltpu.CompilerParams(needs_layout_passes=False),
    )
    return run(p_t, idx_t)


@jax.jit
def kernel(hidden_states, weight, bias):
    x = hidden_states.reshape(-1, HIDDEN_DIM)
    b2 = bias.reshape(NUM_EXPERTS, 1)
    p_t, idx_t = _tc_logits(x, weight, b2)
    s_t = _sc_route(p_t, idx_t)
    return s_t.T, idx_t.T
